# baseline (speedup 1.0000x reference)
import functools

import jax
import jax.numpy as jnp
from jax import lax
from jax.experimental import pallas as pl
from jax.experimental.pallas import tpu as pltpu
from jax.experimental.pallas import tpu_sc as plsc

B = 16384
F = 16
NC = 2
NS = 16
NW = NC * NS
BPW = B // NW
CHUNK = 128
NCHUNK = BPW // CHUNK
RPK = 8
NROWS = 1000000
PROWS = NROWS // RPK

TCH = 2048
NTCH = 489
LAST_W = NROWS - (NTCH - 1) * TCH


def _pack_kernel(src_hbm, tail_hbm, dst_hbm, buf0, buf1, tbuf, obuf,
                 sem0, sem1, semw):
    wid = lax.axis_index("s") * NC + lax.axis_index("c")
    lanes = lax.iota(jnp.int32, F)

    bufs = (buf0, buf1)
    sems = (sem0, sem1)
    copies = [None, None]

    def start_of(k):
        return pl.multiple_of((k * NW + wid) * TCH, 128)

    NK = NTCH // NW

    def repack(kbuf, prows):
        def row_body(pr, cvec):
            prow = cvec >> 3
            for s in range(RPK):
                v = plsc.load_gather(kbuf, [lanes, cvec + s])
                plsc.store_scatter(obuf, [prow, s * F + lanes], v)
            return cvec + RPK

        lax.fori_loop(0, prows, row_body, jnp.zeros((F,), jnp.int32),
                      unroll=4)

    def write_out(k):
        return pltpu.async_copy(
            obuf,
            dst_hbm.at[pl.ds(pl.multiple_of((k * NW + wid) * (TCH // RPK), 8),
                             TCH // RPK), :],
            semw)

    copies[0] = pltpu.async_copy(
        src_hbm.at[:, pl.ds(start_of(0), TCH)], bufs[0], sems[0])
    wcopy = None
    for k in range(NK):
        nxt = k + 1
        if nxt < NK:
            copies[nxt % 2] = pltpu.async_copy(
                src_hbm.at[:, pl.ds(start_of(nxt), TCH)],
                bufs[nxt % 2], sems[nxt % 2])
        copies[k % 2].wait()
        if wcopy is not None:
            wcopy.wait()
        repack(bufs[k % 2], TCH // RPK)
        wcopy = write_out(k)
    wcopy.wait()

    @pl.when(wid < NW - 24)
    def _left_full():
        pltpu.sync_copy(src_hbm.at[:, pl.ds(start_of(NK), TCH)], buf0)
        repack(buf0, TCH // RPK)
        pltpu.sync_copy(
            obuf,
            dst_hbm.at[pl.ds(pl.multiple_of((NK * NW + wid) * (TCH // RPK), 8),
                             TCH // RPK), :])

    @pl.when(wid == 8)
    def _tail():
        c = NTCH - 1
        pltpu.sync_copy(
            src_hbm.at[:, pl.ds(pl.multiple_of(c * TCH, 128), 512)],
            buf1.at[:, pl.ds(0, 512)])
        repack(buf1, 512 // RPK)
        pltpu.sync_copy(
            obuf.at[pl.ds(0, 512 // RPK), :],
            dst_hbm.at[pl.ds(pl.multiple_of(c * (TCH // RPK), 8),
                             512 // RPK), :])

    @pl.when(wid == 9)
    def _tail64():
        pltpu.sync_copy(tail_hbm, tbuf)

        def row_body(pr, carry):
            base = pr * RPK
            prow = jnp.zeros((F,), jnp.int32) + pr
            for s in range(RPK):
                v = plsc.load_gather(
                    tbuf, [lanes, jnp.zeros((F,), jnp.int32) + (base + s)])
                plsc.store_scatter(obuf, [prow, s * F + lanes], v)
            return carry

        lax.fori_loop(0, 64 // RPK, row_body, 0)
        pltpu.sync_copy(
            obuf.at[pl.ds(0, 64 // RPK), :],
            dst_hbm.at[pl.ds(pl.multiple_of(PROWS - 64 // RPK, 8),
                             64 // RPK), :])


def _mf_kernel(uj_hbm, ij_hbm, ur_hbm, ir_hbm, user_hbm, item_hbm, out_hbm,
               u_idx, i_idx, u_rem, i_rem, u_rows0, u_rows1, i_rows0, i_rows1,
               out_v, sem_u, sem_i):
    wid = lax.axis_index("s") * NC + lax.axis_index("c")

    pltpu.sync_copy(uj_hbm.at[wid], u_idx)
    pltpu.sync_copy(ij_hbm.at[wid], i_idx)
    pltpu.sync_copy(ur_hbm.at[wid], u_rem)
    pltpu.sync_copy(ir_hbm.at[wid], i_rem)

    u_bufs = (u_rows0, u_rows1)
    i_bufs = (i_rows0, i_rows1)

    def fire(c):
        uc = pltpu.async_copy(user_hbm.at[u_idx.at[c]], u_bufs[c % 2], sem_u)
        ic = pltpu.async_copy(item_hbm.at[i_idx.at[c]], i_bufs[c % 2], sem_i)
        return uc, ic

    lanes = lax.iota(jnp.int32, F)
    pend = fire(0)
    for c in range(NCHUNK):
        nxt_pend = fire(c + 1) if c + 1 < NCHUNK else None
        pend[0].wait()
        pend[1].wait()
        ub = u_bufs[c % 2]
        ib = i_bufs[c % 2]

        def body(g, carry, c=c, ub=ub, ib=ib):
            rows16 = g * F + lanes
            ucol = u_rem[c, pl.ds(g * F, F)] * F
            icol = i_rem[c, pl.ds(g * F, F)] * F
            acc0 = (plsc.load_gather(ub, [rows16, ucol]) *
                    plsc.load_gather(ib, [rows16, icol]))
            acc1 = (plsc.load_gather(ub, [rows16, ucol + 1]) *
                    plsc.load_gather(ib, [rows16, icol + 1]))
            for f in range(2, F, 2):
                acc0 = acc0 + (plsc.load_gather(ub, [rows16, ucol + f]) *
                               plsc.load_gather(ib, [rows16, icol + f]))
                acc1 = acc1 + (plsc.load_gather(ub, [rows16, ucol + f + 1]) *
                               plsc.load_gather(ib, [rows16, icol + f + 1]))
            out_v[c, pl.ds(g * F, F)] = acc0 + acc1
            return carry

        lax.fori_loop(0, CHUNK // F, body, 0)
        pend = nxt_pend

    pltpu.sync_copy(out_v, out_hbm.at[wid])


@jax.jit
def kernel(u, i, user_emb, item_emb):
    u32 = u.astype(jnp.int32)
    i32 = i.astype(jnp.int32)
    uj = (u32 // RPK).reshape(NW, NCHUNK, CHUNK)
    ij = (i32 // RPK).reshape(NW, NCHUNK, CHUNK)
    ur = (u32 % RPK).reshape(NW, NCHUNK, CHUNK)
    ir = (i32 % RPK).reshape(NW, NCHUNK, CHUNK)

    mesh = plsc.VectorSubcoreMesh(core_axis_name="c", subcore_axis_name="s")
    cp = pltpu.CompilerParams(
        needs_layout_passes=False, use_tc_tiling_on_sc=True,
        disable_bounds_checks=True)

    pack = functools.partial(
        pl.kernel,
        out_type=jax.ShapeDtypeStruct((PROWS, RPK * F), jnp.float32),
        mesh=mesh,
        compiler_params=cp,
        scratch_types=[
            pltpu.VMEM((F, TCH), jnp.float32),
            pltpu.VMEM((F, TCH), jnp.float32),
            pltpu.VMEM((F, 64), jnp.float32),
            pltpu.VMEM((TCH // RPK, RPK * F), jnp.float32),
            pltpu.SemaphoreType.DMA,
            pltpu.SemaphoreType.DMA,
            pltpu.SemaphoreType.DMA,
        ],
    )(_pack_kernel)
    user_t = user_emb.T
    item_t = item_emb.T
    user_p = pack(user_t, user_t[:, NROWS - 64:])
    item_p = pack(item_t, item_t[:, NROWS - 64:])

    k = functools.partial(
        pl.kernel,
        out_type=jax.ShapeDtypeStruct((NW, NCHUNK, CHUNK), jnp.float32),
        mesh=mesh,
        compiler_params=cp,
        scratch_types=[
            pltpu.VMEM((NCHUNK, CHUNK), jnp.int32),
            pltpu.VMEM((NCHUNK, CHUNK), jnp.int32),
            pltpu.VMEM((NCHUNK, CHUNK), jnp.int32),
            pltpu.VMEM((NCHUNK, CHUNK), jnp.int32),
            pltpu.VMEM((CHUNK, RPK * F), jnp.float32),
            pltpu.VMEM((CHUNK, RPK * F), jnp.float32),
            pltpu.VMEM((CHUNK, RPK * F), jnp.float32),
            pltpu.VMEM((CHUNK, RPK * F), jnp.float32),
            pltpu.VMEM((NCHUNK, CHUNK), jnp.float32),
            pltpu.SemaphoreType.DMA,
            pltpu.SemaphoreType.DMA,
        ],
    )(_mf_kernel)
    out = k(uj, ij, ur, ir, user_p, item_p)
    return out.reshape(B)


# speedup vs baseline: 1.4773x; 1.4773x over previous
import functools

import jax
import jax.numpy as jnp
from jax import lax
from jax.experimental import pallas as pl
from jax.experimental.pallas import tpu as pltpu
from jax.experimental.pallas import tpu_sc as plsc

B = 16384
F = 16
NC = 2
NS = 16
NW = NC * NS
BPW = B // NW
EC = 16
NCHUNK = BPW // EC
RPK = 8


def _mf_kernel(uj_hbm, ij_hbm, ur_hbm, ir_hbm, user_hbm, item_hbm, out_hbm,
               u_vst, i_vst, u_rem, i_rem, u_rows, i_rows,
               out_v, sem_u, sem_i):
    wid = lax.axis_index("s") * NC + lax.axis_index("c")

    pltpu.sync_copy(uj_hbm.at[wid], u_vst)
    pltpu.sync_copy(ij_hbm.at[wid], i_vst)
    pltpu.sync_copy(ur_hbm.at[wid], u_rem)
    pltpu.sync_copy(ir_hbm.at[wid], i_rem)

    lanes = lax.iota(jnp.int32, F)
    lanes8 = lanes * RPK

    def chunk(c, carry):
        uvec = u_vst[pl.ds(c * EC, EC)]
        ivec = i_vst[pl.ds(c * EC, EC)]
        for e in range(EC):
            ru = pl.multiple_of(uvec[e], RPK)
            pltpu.async_copy(user_hbm.at[pl.ds(ru, RPK), :],
                             u_rows.at[pl.ds(e * RPK, RPK), :], sem_u)
            ri = pl.multiple_of(ivec[e], RPK)
            pltpu.async_copy(item_hbm.at[pl.ds(ri, RPK), :],
                             i_rows.at[pl.ds(e * RPK, RPK), :], sem_i)
        pltpu.make_async_copy(
            user_hbm.at[pl.ds(0, EC * RPK), :], u_rows, sem_u).wait()
        pltpu.make_async_copy(
            item_hbm.at[pl.ds(0, EC * RPK), :], i_rows, sem_i).wait()

        urows16 = lanes8 + u_rem[pl.ds(c * EC, EC)]
        irows16 = lanes8 + i_rem[pl.ds(c * EC, EC)]
        z = jnp.zeros((F,), jnp.int32)
        acc0 = (plsc.load_gather(u_rows, [urows16, z]) *
                plsc.load_gather(i_rows, [irows16, z]))
        acc1 = (plsc.load_gather(u_rows, [urows16, z + 1]) *
                plsc.load_gather(i_rows, [irows16, z + 1]))
        for f in range(2, F, 2):
            acc0 = acc0 + (plsc.load_gather(u_rows, [urows16, z + f]) *
                           plsc.load_gather(i_rows, [irows16, z + f]))
            acc1 = acc1 + (plsc.load_gather(u_rows, [urows16, z + f + 1]) *
                           plsc.load_gather(i_rows, [irows16, z + f + 1]))
        out_v[pl.ds(c * EC, EC)] = acc0 + acc1
        return carry

    lax.fori_loop(0, NCHUNK, chunk, 0)
    pltpu.sync_copy(out_v, out_hbm.at[wid])


@jax.jit
def kernel(u, i, user_emb, item_emb):
    u32 = u.astype(jnp.int32)
    i32 = i.astype(jnp.int32)
    uj8 = (u32 & ~(RPK - 1)).reshape(NW, BPW)
    ij8 = (i32 & ~(RPK - 1)).reshape(NW, BPW)
    ur = (u32 % RPK).reshape(NW, BPW)
    ir = (i32 % RPK).reshape(NW, BPW)

    mesh = plsc.VectorSubcoreMesh(core_axis_name="c", subcore_axis_name="s")
    cp = pltpu.CompilerParams(
        needs_layout_passes=False, use_tc_tiling_on_sc=True,
        disable_bounds_checks=True)

    k = functools.partial(
        pl.kernel,
        out_type=jax.ShapeDtypeStruct((NW, BPW), jnp.float32),
        mesh=mesh,
        compiler_params=cp,
        scratch_types=[
            pltpu.VMEM((BPW,), jnp.int32),
            pltpu.VMEM((BPW,), jnp.int32),
            pltpu.VMEM((BPW,), jnp.int32),
            pltpu.VMEM((BPW,), jnp.int32),
            pltpu.VMEM((EC * RPK, F), jnp.float32),
            pltpu.VMEM((EC * RPK, F), jnp.float32),
            pltpu.VMEM((BPW,), jnp.float32),
            pltpu.SemaphoreType.DMA,
            pltpu.SemaphoreType.DMA,
        ],
    )(_mf_kernel)
    out = k(uj8, ij8, ur, ir, user_emb, item_emb)
    return out.reshape(B)


# speedup vs baseline: 2.5018x; 1.6935x over previous
import functools

import jax
import jax.numpy as jnp
from jax import lax
from jax.experimental import pallas as pl
from jax.experimental.pallas import tpu as pltpu
from jax.experimental.pallas import tpu_sc as plsc

B = 16384
F = 16
NC = 2
NS = 16
NW = NC * NS
BPW = B // NW
EC = 16
NCHUNK = BPW // EC
RPK = 8


def _mf_kernel(uj_hbm, ij_hbm, ur_hbm, ir_hbm, user_hbm, item_hbm, out_hbm,
               u_vst, i_vst, u_rem, i_rem, u_rows, i_rows,
               out_v, sem_u, sem_i):
    wid = lax.axis_index("s") * NC + lax.axis_index("c")

    pltpu.sync_copy(uj_hbm.at[wid], u_vst)
    pltpu.sync_copy(ij_hbm.at[wid], i_vst)
    pltpu.sync_copy(ur_hbm.at[wid], u_rem)
    pltpu.sync_copy(ir_hbm.at[wid], i_rem)

    lanes = lax.iota(jnp.int32, F)
    lanes8 = lanes * RPK

    def chunk(c, carry):
        uvec = u_vst[pl.ds(c * EC, EC)]
        ivec = i_vst[pl.ds(c * EC, EC)]
        copies = []
        for e in range(EC):
            copies.append(pltpu.async_copy(
                user_hbm.at[uvec[e]],
                u_rows.at[pl.ds(e * RPK, RPK), :], sem_u))
            copies.append(pltpu.async_copy(
                item_hbm.at[ivec[e]],
                i_rows.at[pl.ds(e * RPK, RPK), :], sem_i))
        for cp_ in copies:
            cp_.wait()

        urows16 = lanes8 + u_rem[pl.ds(c * EC, EC)]
        irows16 = lanes8 + i_rem[pl.ds(c * EC, EC)]
        z = jnp.zeros((F,), jnp.int32)
        acc0 = (plsc.load_gather(u_rows, [urows16, z]) *
                plsc.load_gather(i_rows, [irows16, z]))
        acc1 = (plsc.load_gather(u_rows, [urows16, z + 1]) *
                plsc.load_gather(i_rows, [irows16, z + 1]))
        for f in range(2, F, 2):
            acc0 = acc0 + (plsc.load_gather(u_rows, [urows16, z + f]) *
                           plsc.load_gather(i_rows, [irows16, z + f]))
            acc1 = acc1 + (plsc.load_gather(u_rows, [urows16, z + f + 1]) *
                           plsc.load_gather(i_rows, [irows16, z + f + 1]))
        out_v[pl.ds(c * EC, EC)] = acc0 + acc1
        return carry

    lax.fori_loop(0, NCHUNK, chunk, 0)
    pltpu.sync_copy(out_v, out_hbm.at[wid])


@jax.jit
def kernel(u, i, user_emb, item_emb):
    u32 = u.astype(jnp.int32)
    i32 = i.astype(jnp.int32)
    uj8 = (u32 // RPK).reshape(NW, BPW)
    ij8 = (i32 // RPK).reshape(NW, BPW)
    ur = (u32 % RPK).reshape(NW, BPW)
    ir = (i32 % RPK).reshape(NW, BPW)
    user3 = user_emb.reshape(1000000 // RPK, RPK, F)
    item3 = item_emb.reshape(1000000 // RPK, RPK, F)

    mesh = plsc.VectorSubcoreMesh(core_axis_name="c", subcore_axis_name="s")
    cp = pltpu.CompilerParams(
        needs_layout_passes=False, use_tc_tiling_on_sc=True,
        disable_bounds_checks=True)

    k = functools.partial(
        pl.kernel,
        out_type=jax.ShapeDtypeStruct((NW, BPW), jnp.float32),
        mesh=mesh,
        compiler_params=cp,
        scratch_types=[
            pltpu.VMEM((BPW,), jnp.int32),
            pltpu.VMEM((BPW,), jnp.int32),
            pltpu.VMEM((BPW,), jnp.int32),
            pltpu.VMEM((BPW,), jnp.int32),
            pltpu.VMEM((EC * RPK, F), jnp.float32),
            pltpu.VMEM((EC * RPK, F), jnp.float32),
            pltpu.VMEM((BPW,), jnp.float32),
            pltpu.SemaphoreType.DMA,
            pltpu.SemaphoreType.DMA,
        ],
    )(_mf_kernel)
    out = k(uj8, ij8, ur, ir, user3, item3)
    return out.reshape(B)
